# baseline (speedup 1.0000x reference)
import functools

import jax
import jax.numpy as jnp
from jax import lax
from jax.experimental import pallas as pl
from jax.experimental.pallas import tpu as pltpu
from jax.experimental.pallas import tpu_sc as plsc

VOCAB = 1000000
EMB = 64
HID = 256
B = 4096
L = 200

NC = 2
NS = 16
NW = NC * NS
ROWS_PER_W = B // NW
L_PAD = 208
NVREG = L_PAD // 16
NBUF = 2


def _sc_gather_sum(idx2, table):
    mesh = plsc.VectorSubcoreMesh(core_axis_name="c", subcore_axis_name="s")

    @functools.partial(
        pl.kernel,
        mesh=mesh,
        out_type=jax.ShapeDtypeStruct((B, EMB), jnp.float32),
        scratch_types=[
            pltpu.VMEM((ROWS_PER_W, L_PAD), jnp.int32),
            pltpu.VMEM((NBUF, L_PAD, 2 * EMB), jnp.float32),
            pltpu.VMEM((ROWS_PER_W, EMB), jnp.float32),
            [pltpu.SemaphoreType.DMA] * NBUF,
        ],
    )
    def k(idx_hbm, table_hbm, out_hbm, idx_v, rows_v, out_v, sems):
        wid = lax.axis_index("s") * NC + lax.axis_index("c")
        base = wid * ROWS_PER_W

        tview = table_hbm

        pltpu.sync_copy(idx_hbm.at[pl.ds(base, ROWS_PER_W)], idx_v)

        def issue(row, b):
            for kk in range(NVREG):
                iv = idx_v[row, pl.ds(16 * kk, 16)]
                pltpu.async_copy(
                    tview.at[iv],
                    rows_v.at[b, pl.ds(16 * kk, 16)],
                    sems[b],
                )

        def wait(b):
            pltpu.make_async_copy(
                tview.at[pl.ds(0, L_PAD)], rows_v.at[b], sems[b]
            ).wait()

        def accum(row, b):
            zero = jnp.zeros((16,), jnp.float32)

            @plsc.parallel_loop(0, L, unroll=8, carry=(zero,) * 4)
            def accs(j, c):
                a0, a1, a2, a3 = c
                a0 = a0 + rows_v[b, j, pl.ds(0, 16)]
                a1 = a1 + rows_v[b, j, pl.ds(16, 16)]
                a2 = a2 + rows_v[b, j, pl.ds(32, 16)]
                a3 = a3 + rows_v[b, j, pl.ds(48, 16)]
                return (a0, a1, a2, a3)

            for g in range(4):
                out_v[row, pl.ds(16 * g, 16)] = accs[g]

        for b in range(NBUF):
            issue(b, b)

        def outer(i, carry):
            for b in range(NBUF):
                row = NBUF * i + b
                wait(b)
                nxt = row + NBUF

                @pl.when(nxt < ROWS_PER_W)
                def _():
                    issue(nxt, b)

                accum(row, b)
            return carry

        lax.fori_loop(0, ROWS_PER_W // NBUF, outer, 0)

        pltpu.sync_copy(out_v, out_hbm.at[pl.ds(base, ROWS_PER_W)])

    return k(idx2, table)


def _mlp_body(x_ref, w1_ref, b1_ref, w2_ref, b2_ref, o_ref):
    x = x_ref[...] * jnp.float32(1.0 / L)
    h = jnp.dot(x, w1_ref[...], preferred_element_type=jnp.float32,
                precision=lax.Precision.HIGHEST)
    h = jnp.maximum(h + b1_ref[...], 0.0)
    z = jnp.dot(h, w2_ref[...], preferred_element_type=jnp.float32,
                precision=lax.Precision.HIGHEST)
    z = z + b2_ref[...]
    m = jnp.max(z, axis=1, keepdims=True)
    s = z - m
    lse = jnp.log(jnp.sum(jnp.exp(s), axis=1, keepdims=True))
    o_ref[...] = s - lse


def _mlp(sums, W1, b1, W2, b2):
    W2p = jnp.pad(W2, ((0, 0), (0, 128 - W2.shape[1])))
    b2p = jnp.pad(b2, (0, 128 - b2.shape[0]), constant_values=-1e30)
    grid = 4
    blk = B // grid
    out = pl.pallas_call(
        _mlp_body,
        grid=(grid,),
        in_specs=[
            pl.BlockSpec((blk, EMB), lambda i: (i, 0)),
            pl.BlockSpec((EMB, HID), lambda i: (0, 0)),
            pl.BlockSpec((1, HID), lambda i: (0, 0)),
            pl.BlockSpec((HID, 128), lambda i: (0, 0)),
            pl.BlockSpec((1, 128), lambda i: (0, 0)),
        ],
        out_specs=pl.BlockSpec((blk, 128), lambda i: (i, 0)),
        out_shape=jax.ShapeDtypeStruct((B, 128), jnp.float32),
    )(sums, W1, b1.reshape(1, HID), W2p, b2p.reshape(1, 128))
    return out[:, :2]


def kernel(input_idxs, table, W1, b1, W2, b2):
    idx2 = jnp.pad(input_idxs.astype(jnp.int32), ((0, 0), (0, L_PAD - L)))
    tpad = jnp.pad(table, ((0, 0), (0, EMB)))
    sums = _sc_gather_sum(idx2, tpad)
    return _mlp(sums, W1, b1, W2, b2)


# speedup vs baseline: 3.3031x; 3.3031x over previous
import functools

import jax
import jax.numpy as jnp
from jax import lax
from jax.experimental import pallas as pl
from jax.experimental.pallas import tpu as pltpu
from jax.experimental.pallas import tpu_sc as plsc

VOCAB = 1000000
EMB = 64
HID = 256
B = 4096
L = 200

NC = 2
NS = 16
NW = NC * NS
ROWS_PER_W = B // NW
NBUF = 2


def _sc_gather_sum(idx2, table):
    mesh = plsc.VectorSubcoreMesh(core_axis_name="c", subcore_axis_name="s")

    @functools.partial(
        pl.kernel,
        mesh=mesh,
        out_type=jax.ShapeDtypeStruct((B, EMB), jnp.float32),
        scratch_types=[
            pltpu.VMEM((ROWS_PER_W, L), jnp.int32),
            pltpu.VMEM((NBUF, L, EMB), jnp.float32),
            pltpu.VMEM((ROWS_PER_W, EMB), jnp.float32),
            [pltpu.SemaphoreType.DMA] * NBUF,
        ],
    )
    def k(idx_hbm, table_hbm, out_hbm, idx_v, rows_v, out_v, sems):
        wid = lax.axis_index("s") * NC + lax.axis_index("c")
        base = wid * ROWS_PER_W

        pltpu.sync_copy(idx_hbm.at[pl.ds(base, ROWS_PER_W)], idx_v)

        def issue_block(row, b, k):
            vec = idx_v[row, pl.ds(16 * k, 16)]
            for c in range(16):
                t = vec[c]
                j = 16 * k + c
                pltpu.async_copy(
                    table_hbm.at[pl.ds(t, 1)],
                    rows_v.at[b, pl.ds(j, 1)],
                    sems[b],
                )

        def issue(row, b):
            for k in range(L // 16):
                issue_block(row, b, k)
            vec = idx_v[row, pl.ds(L - 16, 16)]
            for c in range(8, 16):
                t = vec[c]
                j = L - 16 + c
                pltpu.async_copy(
                    table_hbm.at[pl.ds(t, 1)],
                    rows_v.at[b, pl.ds(j, 1)],
                    sems[b],
                )

        def wait(b):
            pltpu.make_async_copy(
                table_hbm.at[pl.ds(0, L)], rows_v.at[b], sems[b]
            ).wait()

        def accum_16(b, k, accs):
            for cc in range(16):
                j = 16 * k + cc
                for g in range(4):
                    accs[g] = accs[g] + rows_v[b, j, pl.ds(16 * g, 16)]
            return accs

        def accum_body(row, b, nxt, bn):
            zero = jnp.zeros((16,), jnp.float32)

            @plsc.parallel_loop(0, L // 16, unroll=1, carry=(zero,) * 4)
            def accs(k, c):
                if nxt is not None:
                    issue_block(nxt, bn, k)
                return tuple(accum_16(b, k, list(c)))

            accs = list(accs)
            if nxt is not None:
                vec = idx_v[nxt, pl.ds(L - 16, 16)]
                for c in range(8, 16):
                    t = vec[c]
                    j = L - 16 + c
                    pltpu.async_copy(
                        table_hbm.at[pl.ds(t, 1)],
                        rows_v.at[bn, pl.ds(j, 1)],
                        sems[bn],
                    )
            for j in range(L - 8, L):
                for g in range(4):
                    accs[g] = accs[g] + rows_v[b, j, pl.ds(16 * g, 16)]

            for g in range(4):
                out_v[row, pl.ds(16 * g, 16)] = accs[g]

        issue(0, 0)

        def outer(i, carry):
            for b in range(NBUF):
                row = NBUF * i + b
                wait(b)
                nxt = row + 1
                bn = (b + 1) % NBUF

                @pl.when(nxt < ROWS_PER_W)
                def _():
                    accum_body(row, b, nxt, bn)

                @pl.when(nxt >= ROWS_PER_W)
                def _():
                    accum_body(row, b, None, None)
            return carry

        lax.fori_loop(0, ROWS_PER_W // NBUF, outer, 0)

        pltpu.sync_copy(out_v, out_hbm.at[pl.ds(base, ROWS_PER_W)])

    return k(idx2, table)


def _mlp_body(x_ref, w1_ref, b1_ref, w2_ref, b2_ref, o_ref):
    x = x_ref[...] * jnp.float32(1.0 / L)
    h = jnp.dot(x, w1_ref[...], preferred_element_type=jnp.float32,
                precision=lax.Precision.HIGHEST)
    h = jnp.maximum(h + b1_ref[...], 0.0)
    z = jnp.dot(h, w2_ref[...], preferred_element_type=jnp.float32,
                precision=lax.Precision.HIGHEST)
    z = z + b2_ref[...]
    m = jnp.max(z, axis=1, keepdims=True)
    s = z - m
    lse = jnp.log(jnp.sum(jnp.exp(s), axis=1, keepdims=True))
    o_ref[...] = s - lse


def _mlp(sums, W1, b1, W2, b2):
    W2p = jnp.pad(W2, ((0, 0), (0, 128 - W2.shape[1])))
    b2p = jnp.pad(b2, (0, 128 - b2.shape[0]), constant_values=-1e30)
    grid = 4
    blk = B // grid
    out = pl.pallas_call(
        _mlp_body,
        grid=(grid,),
        in_specs=[
            pl.BlockSpec((blk, EMB), lambda i: (i, 0)),
            pl.BlockSpec((EMB, HID), lambda i: (0, 0)),
            pl.BlockSpec((1, HID), lambda i: (0, 0)),
            pl.BlockSpec((HID, 128), lambda i: (0, 0)),
            pl.BlockSpec((1, 128), lambda i: (0, 0)),
        ],
        out_specs=pl.BlockSpec((blk, 128), lambda i: (i, 0)),
        out_shape=jax.ShapeDtypeStruct((B, 128), jnp.float32),
    )(sums, W1, b1.reshape(1, HID), W2p, b2p.reshape(1, 128))
    return out[:, :2]


def kernel(input_idxs, table, W1, b1, W2, b2):
    sums = _sc_gather_sum(input_idxs.astype(jnp.int32), table)
    return _mlp(sums, W1, b1, W2, b2)


# speedup vs baseline: 3.8464x; 1.1645x over previous
import functools

import jax
import jax.numpy as jnp
from jax import lax
from jax.experimental import pallas as pl
from jax.experimental.pallas import tpu as pltpu
from jax.experimental.pallas import tpu_sc as plsc

VOCAB = 1000000
EMB = 64
HID = 256
B = 4096
L = 200

NC = 2
NS = 16
NW = NC * NS
ROWS_PER_W = B // NW
NBUF = 2


def _sc_gather_sum(idx2, table):
    mesh = plsc.VectorSubcoreMesh(core_axis_name="c", subcore_axis_name="s")

    @functools.partial(
        pl.kernel,
        mesh=mesh,
        out_type=jax.ShapeDtypeStruct((B, EMB), jnp.float32),
        scratch_types=[
            pltpu.VMEM((ROWS_PER_W, L), jnp.int32),
            pltpu.VMEM((NBUF, L, EMB), jnp.float32),
            pltpu.VMEM((ROWS_PER_W, EMB), jnp.float32),
            [pltpu.SemaphoreType.DMA] * NBUF,
        ],
    )
    def k(idx_hbm, table_hbm, out_hbm, idx_v, rows_v, out_v, sems):
        wid = lax.axis_index("s") * NC + lax.axis_index("c")
        base = wid * ROWS_PER_W

        pltpu.sync_copy(idx_hbm.at[pl.ds(base, ROWS_PER_W)], idx_v)

        def issue_block(row, b, k):
            vec = idx_v[row, pl.ds(16 * k, 16)]
            for c in range(16):
                t = vec[c]
                j = 16 * k + c
                pltpu.async_copy(
                    table_hbm.at[pl.ds(t, 1)],
                    rows_v.at[b, pl.ds(j, 1)],
                    sems[b],
                )

        def issue(row, b):
            for k in range(L // 16):
                issue_block(row, b, k)
            vec = idx_v[row, pl.ds(L - 16, 16)]
            for c in range(8, 16):
                t = vec[c]
                j = L - 16 + c
                pltpu.async_copy(
                    table_hbm.at[pl.ds(t, 1)],
                    rows_v.at[b, pl.ds(j, 1)],
                    sems[b],
                )

        def wait(b):
            pltpu.make_async_copy(
                table_hbm.at[pl.ds(0, L)], rows_v.at[b], sems[b]
            ).wait()

        def accum_16(b, k, accs):
            for cc in range(16):
                j = 16 * k + cc
                for g in range(4):
                    accs[g] = accs[g] + rows_v[b, j, pl.ds(16 * g, 16)]
            return accs

        def accum_body(row, b, nxt, bn):
            zero = jnp.zeros((16,), jnp.float32)

            @plsc.parallel_loop(0, L // 16, unroll=1, carry=(zero,) * 4)
            def accs(k, c):
                if nxt is not None:
                    issue_block(nxt, bn, k)
                return tuple(accum_16(b, k, list(c)))

            accs = list(accs)
            if nxt is not None:
                vec = idx_v[nxt, pl.ds(L - 16, 16)]
                for c in range(8, 16):
                    t = vec[c]
                    j = L - 16 + c
                    pltpu.async_copy(
                        table_hbm.at[pl.ds(t, 1)],
                        rows_v.at[bn, pl.ds(j, 1)],
                        sems[bn],
                    )
            for j in range(L - 8, L):
                for g in range(4):
                    accs[g] = accs[g] + rows_v[b, j, pl.ds(16 * g, 16)]

            for g in range(4):
                out_v[row, pl.ds(16 * g, 16)] = accs[g]

        issue(0, 0)

        def outer(i, carry):
            for b in range(NBUF):
                row = NBUF * i + b
                wait(b)
                nxt = row + 1
                bn = (b + 1) % NBUF

                @pl.when(nxt < ROWS_PER_W)
                def _():
                    accum_body(row, b, nxt, bn)

                @pl.when(nxt >= ROWS_PER_W)
                def _():
                    accum_body(row, b, None, None)
            return carry

        lax.fori_loop(0, ROWS_PER_W // NBUF, outer, 0)

        pltpu.sync_copy(out_v, out_hbm.at[pl.ds(base, ROWS_PER_W)])

    return k(idx2, table)


def _mlp_body(x_ref, w1_ref, b1_ref, w2_ref, b2_ref, o_ref):
    x = x_ref[...] * jnp.float32(1.0 / L)
    h = jnp.dot(x, w1_ref[...], preferred_element_type=jnp.float32,
                precision=lax.Precision.HIGHEST)
    h = jnp.maximum(h + b1_ref[...], 0.0)
    z = jnp.dot(h, w2_ref[...], preferred_element_type=jnp.float32,
                precision=lax.Precision.HIGHEST)
    z = z + b2_ref[...]
    m = jnp.max(z, axis=1, keepdims=True)
    s = z - m
    lse = jnp.log(jnp.sum(jnp.exp(s), axis=1, keepdims=True))
    o_ref[...] = s - lse


def _mlp(sums, W1, b1, W2, b2):
    W2p = jnp.pad(W2, ((0, 0), (0, 128 - W2.shape[1])))
    b2p = jnp.pad(b2, (0, 128 - b2.shape[0]), constant_values=-1e30)
    grid = 4
    blk = B // grid
    out = pl.pallas_call(
        _mlp_body,
        grid=(grid,),
        in_specs=[
            pl.BlockSpec((blk, EMB), lambda i: (i, 0)),
            pl.BlockSpec((EMB, HID), lambda i: (0, 0)),
            pl.BlockSpec((1, HID), lambda i: (0, 0)),
            pl.BlockSpec((HID, 128), lambda i: (0, 0)),
            pl.BlockSpec((1, 128), lambda i: (0, 0)),
        ],
        out_specs=pl.BlockSpec((blk, 128), lambda i: (i, 0)),
        out_shape=jax.ShapeDtypeStruct((B, 128), jnp.float32),
    )(sums, W1, b1.reshape(1, HID), W2p, b2p.reshape(1, 128))
    return out[:, :2]


def _tr_body(x_ref, o_ref):
    o_ref[...] = x_ref[...].T


def _tc_transpose(tableT):
    blk = 12800
    grid = pl.cdiv(VOCAB, blk)
    return pl.pallas_call(
        _tr_body,
        grid=(grid,),
        in_specs=[pl.BlockSpec((EMB, blk), lambda i: (0, i))],
        out_specs=pl.BlockSpec((blk, EMB), lambda i: (i, 0)),
        out_shape=jax.ShapeDtypeStruct((VOCAB, EMB), jnp.float32),
    )(tableT)


def kernel(input_idxs, table, W1, b1, W2, b2):
    tbl = _tc_transpose(table.T)
    sums = _sc_gather_sum(input_idxs.astype(jnp.int32), tbl)
    return _mlp(sums, W1, b1, W2, b2)


# speedup vs baseline: 3.9141x; 1.0176x over previous
import functools

import jax
import jax.numpy as jnp
from jax import lax
from jax.experimental import pallas as pl
from jax.experimental.pallas import tpu as pltpu
from jax.experimental.pallas import tpu_sc as plsc

VOCAB = 1000000
EMB = 64
HID = 256
B = 4096
L = 200

NC = 2
NS = 16
NW = NC * NS
ROWS_PER_W = B // NW
NBUF = 2


def _sc_gather_sum(idx2, table):
    mesh = plsc.VectorSubcoreMesh(core_axis_name="c", subcore_axis_name="s")

    @functools.partial(
        pl.kernel,
        mesh=mesh,
        out_type=jax.ShapeDtypeStruct((B, EMB), jnp.float32),
        scratch_types=[
            pltpu.VMEM((ROWS_PER_W, L), jnp.int32),
            pltpu.VMEM((NBUF, L, EMB), jnp.float32),
            pltpu.VMEM((ROWS_PER_W, EMB), jnp.float32),
            [pltpu.SemaphoreType.DMA] * NBUF,
        ],
    )
    def k(idx_hbm, table_hbm, out_hbm, idx_v, rows_v, out_v, sems):
        wid = lax.axis_index("s") * NC + lax.axis_index("c")
        base = wid * ROWS_PER_W

        pltpu.sync_copy(idx_hbm.at[pl.ds(base, ROWS_PER_W)], idx_v)

        def issue_block(row, b, k):
            vec = idx_v[row, pl.ds(16 * k, 16)]
            for c in range(16):
                t = vec[c]
                j = 16 * k + c
                pltpu.async_copy(
                    table_hbm.at[pl.ds(t, 1)],
                    rows_v.at[b, pl.ds(j, 1)],
                    sems[b],
                )

        def issue(row, b):
            for k in range(L // 16):
                issue_block(row, b, k)
            vec = idx_v[row, pl.ds(L - 16, 16)]
            for c in range(8, 16):
                t = vec[c]
                j = L - 16 + c
                pltpu.async_copy(
                    table_hbm.at[pl.ds(t, 1)],
                    rows_v.at[b, pl.ds(j, 1)],
                    sems[b],
                )

        def wait(b):
            pltpu.make_async_copy(
                table_hbm.at[pl.ds(0, L)], rows_v.at[b], sems[b]
            ).wait()

        def accum_16(b, k, accs):
            for cc in range(16):
                j = 16 * k + cc
                for g in range(4):
                    accs[g] = accs[g] + rows_v[b, j, pl.ds(16 * g, 16)]
            return accs

        def accum_body(row, b, nxt, bn):
            zero = jnp.zeros((16,), jnp.float32)

            @plsc.parallel_loop(0, L // 16, unroll=1, carry=(zero,) * 4)
            def accs(k, c):
                if nxt is not None:
                    issue_block(nxt, bn, k)
                return tuple(accum_16(b, k, list(c)))

            accs = list(accs)
            if nxt is not None:
                vec = idx_v[nxt, pl.ds(L - 16, 16)]
                for c in range(8, 16):
                    t = vec[c]
                    j = L - 16 + c
                    pltpu.async_copy(
                        table_hbm.at[pl.ds(t, 1)],
                        rows_v.at[bn, pl.ds(j, 1)],
                        sems[bn],
                    )
            for j in range(L - 8, L):
                for g in range(4):
                    accs[g] = accs[g] + rows_v[b, j, pl.ds(16 * g, 16)]

            for g in range(4):
                out_v[row, pl.ds(16 * g, 16)] = accs[g]

        issue(0, 0)

        def outer(i, carry):
            for b in range(NBUF):
                row = NBUF * i + b
                wait(b)
                nxt = row + 1
                bn = (b + 1) % NBUF

                @pl.when(nxt < ROWS_PER_W)
                def _():
                    accum_body(row, b, nxt, bn)

                @pl.when(nxt >= ROWS_PER_W)
                def _():
                    accum_body(row, b, None, None)
            return carry

        lax.fori_loop(0, ROWS_PER_W // NBUF, outer, 0)

        pltpu.sync_copy(out_v, out_hbm.at[pl.ds(base, ROWS_PER_W)])

    return k(idx2, table)


def _mlp_body(x_ref, w1_ref, b1_ref, w2_ref, b2_ref, o_ref):
    x = x_ref[...] * jnp.float32(1.0 / L)
    h = jnp.dot(x, w1_ref[...], preferred_element_type=jnp.float32,
                precision=lax.Precision.HIGHEST)
    h = jnp.maximum(h + b1_ref[...], 0.0)
    z = jnp.dot(h, w2_ref[...], preferred_element_type=jnp.float32,
                precision=lax.Precision.HIGHEST)
    z = z + b2_ref[...]
    m = jnp.max(z, axis=1, keepdims=True)
    s = z - m
    lse = jnp.log(jnp.sum(jnp.exp(s), axis=1, keepdims=True))
    o_ref[...] = s - lse


def _mlp(sums, W1, b1, W2, b2):
    W2p = jnp.pad(W2, ((0, 0), (0, 128 - W2.shape[1])))
    b2p = jnp.pad(b2, (0, 128 - b2.shape[0]), constant_values=-1e30)
    grid = 4
    blk = B // grid
    out = pl.pallas_call(
        _mlp_body,
        grid=(grid,),
        in_specs=[
            pl.BlockSpec((blk, EMB), lambda i: (i, 0)),
            pl.BlockSpec((EMB, HID), lambda i: (0, 0)),
            pl.BlockSpec((1, HID), lambda i: (0, 0)),
            pl.BlockSpec((HID, 128), lambda i: (0, 0)),
            pl.BlockSpec((1, 128), lambda i: (0, 0)),
        ],
        out_specs=pl.BlockSpec((blk, 128), lambda i: (i, 0)),
        out_shape=jax.ShapeDtypeStruct((B, 128), jnp.float32),
    )(sums, W1, b1.reshape(1, HID), W2p, b2p.reshape(1, 128))
    return out[:, :2]


def _tr_body(x_ref, o_ref):
    o_ref[...] = x_ref[...].T


def _tc_transpose(tableT):
    blk = 25600
    grid = pl.cdiv(VOCAB, blk)
    return pl.pallas_call(
        _tr_body,
        grid=(grid,),
        in_specs=[pl.BlockSpec((EMB, blk), lambda i: (0, i))],
        out_specs=pl.BlockSpec((blk, EMB), lambda i: (i, 0)),
        out_shape=jax.ShapeDtypeStruct((VOCAB, EMB), jnp.float32),
    )(tableT)


def kernel(input_idxs, table, W1, b1, W2, b2):
    tbl = _tc_transpose(table.T)
    sums = _sc_gather_sum(input_idxs.astype(jnp.int32), tbl)
    return _mlp(sums, W1, b1, W2, b2)
